# baseline (speedup 1.0000x reference)
import functools

import numpy as np
import jax
import jax.numpy as jnp
from jax import lax
from jax.experimental import pallas as pl
from jax.experimental.pallas import tpu as pltpu
from jax.experimental.pallas import tpu_sc as plsc

LEVELS = 4
NUM_EMB = 8192
EMB_DIM = 256
BETA = 0.25
B = 4096

TB = 256
NB = B // TB
KB = 2048
NKC = NUM_EMB // KB
LOSS_SCALE = (1.0 + BETA) / (B * EMB_DIM)
IMAX = np.int32(2**31 - 1)


def _rowsumsq(x):
    s = x[:, 0:128] * x[:, 0:128] + x[:, 128:256] * x[:, 128:256]
    st = jnp.transpose(s)
    acc = st[0:8, :]
    for g in range(1, 16):
        acc = acc + st[8 * g:8 * g + 8, :]
    t1 = acc + pltpu.roll(acc, 4, 0)
    t2 = t1 + pltpu.roll(t1, 6, 0)
    t3 = t2 + pltpu.roll(t2, 7, 0)
    return t3[0:1, :]


def _search(r, embbf_scr, esq_scr):
    zsql = _rowsumsq(r)
    zb = jnp.broadcast_to(zsql, (8, TB))
    rb = (r * (-2.0)).astype(jnp.bfloat16)
    colmin = None
    grpsel = None
    for kc in range(NKC):
        eb = embbf_scr[KB * kc:KB * (kc + 1), :]
        mt = lax.dot_general(eb, rb, (((1,), (1,)), ((), ())),
                             preferred_element_type=jnp.float32)
        for r8 in range(KB // 8):
            grp = kc * (KB // 8) + r8
            d = (zb + mt[8 * r8:8 * r8 + 8, :]) \
                + esq_scr[8 * grp:8 * grp + 8, :]
            if colmin is None:
                colmin = d
                grpsel = jnp.zeros((8, TB), jnp.int32)
            else:
                grpsel = jnp.where(d < colmin, jnp.int32(grp), grpsel)
                colmin = jnp.minimum(colmin, d)
    m1 = jnp.minimum(colmin, pltpu.roll(colmin, 4, 0))
    m2 = jnp.minimum(m1, pltpu.roll(m1, 6, 0))
    rowmin = jnp.minimum(m2, pltpu.roll(m2, 7, 0))[0:1, :]
    sub = lax.broadcasted_iota(jnp.int32, (8, TB), 0)
    cand = jnp.where(colmin == rowmin, grpsel * 8 + sub, IMAX)
    c1 = jnp.minimum(cand, pltpu.roll(cand, 4, 0))
    c2 = jnp.minimum(c1, pltpu.roll(c1, 6, 0))
    mini = jnp.minimum(c2, pltpu.roll(c2, 7, 0))[0:1, :]
    return rowmin, mini


def _acc_loss(i, lin_ref, lout_ref, minv):
    part = jnp.sum(minv, axis=1, keepdims=True) * LOSS_SCALE

    @pl.when(i == 0)
    def _():
        lout_ref[:, :] = lin_ref[:, :] + part

    @pl.when(i > 0)
    def _():
        lout_ref[:, :] = lout_ref[:, :] + part


def _level0_body(r_ref, embbf_ref, esqb_ref, lin_ref, idx_ref, lout_ref):
    i = pl.program_id(0)
    minv, mini = _search(r_ref[:], embbf_ref, esqb_ref)
    idx_ref[:] = mini.reshape(1, 1, TB)
    _acc_loss(i, lin_ref, lout_ref, minv)


def _leveln_body(has_q, r_ref, zq_ref, q_ref, embbf_ref, esqb_ref, lin_ref,
                 idx_ref, rn_ref, qn_ref, lout_ref):
    i = pl.program_id(0)
    r = r_ref[:]
    diff = zq_ref[:] - r
    c = r + diff
    rn = r - c
    qn = (q_ref[:] + c) if has_q else c
    rn_ref[:] = rn
    qn_ref[:] = qn
    minv, mini = _search(rn, embbf_ref, esqb_ref)
    idx_ref[:] = mini.reshape(1, 1, TB)
    _acc_loss(i, lin_ref, lout_ref, minv)


HB = B // 2
NBH = HB // TB

_BLK_ROWS = pl.BlockSpec((TB, EMB_DIM), lambda i: (i, 0))
_BLK_IDX = pl.BlockSpec((1, 1, TB), lambda i: (i, 0, 0))
_BLK_EMBBF = pl.BlockSpec((NUM_EMB, EMB_DIM), lambda i: (0, 0))
_BLK_ESQB = pl.BlockSpec((NUM_EMB, TB), lambda i: (0, 0))
_BLK_SCALAR = pl.BlockSpec((1, 1), lambda i: (0, 0))


def _prep_body(cb_ref, embbf_ref, esqb_ref):
    cb = cb_ref[:]
    embbf_ref[:] = cb.astype(jnp.bfloat16)
    esqb_ref[:] = jnp.broadcast_to(
        jnp.sum(cb * cb, axis=1, keepdims=True), (NUM_EMB, TB))


def _prep_call(cb):
    return pl.pallas_call(
        _prep_body,
        out_shape=[jax.ShapeDtypeStruct((NUM_EMB, EMB_DIM), jnp.bfloat16),
                   jax.ShapeDtypeStruct((NUM_EMB, TB), jnp.float32)],
    )(cb)


def _level0_call(r, embbf, esqb, lin):
    return pl.pallas_call(
        _level0_body,
        grid=(NBH,),
        in_specs=[_BLK_ROWS, _BLK_EMBBF, _BLK_ESQB, _BLK_SCALAR],
        out_specs=[_BLK_IDX, _BLK_SCALAR],
        out_shape=[jax.ShapeDtypeStruct((NBH, 1, TB), jnp.int32),
                   jax.ShapeDtypeStruct((1, 1), jnp.float32)],
    )(r, embbf, esqb, lin)


def _leveln_call(has_q, r, zq, q, embbf, esqb, lin):
    return pl.pallas_call(
        functools.partial(_leveln_body, has_q),
        grid=(NBH,),
        in_specs=[_BLK_ROWS, _BLK_ROWS, _BLK_ROWS, _BLK_EMBBF, _BLK_ESQB,
                  _BLK_SCALAR],
        out_specs=[_BLK_IDX, _BLK_ROWS, _BLK_ROWS, _BLK_SCALAR],
        out_shape=[jax.ShapeDtypeStruct((NBH, 1, TB), jnp.int32),
                   jax.ShapeDtypeStruct((HB, EMB_DIM), jnp.float32),
                   jax.ShapeDtypeStruct((HB, EMB_DIM), jnp.float32),
                   jax.ShapeDtypeStruct((1, 1), jnp.float32)],
    )(r, zq, q, embbf, esqb, lin)


def _final_body(r_ref, zq_ref, q_ref, out_ref):
    r = r_ref[:]
    c = r + (zq_ref[:] - r)
    out_ref[:] = q_ref[:] + c


def _final_call(r, zq, q):
    return pl.pallas_call(
        _final_body,
        grid=(NBH,),
        in_specs=[_BLK_ROWS, _BLK_ROWS, _BLK_ROWS],
        out_specs=_BLK_ROWS,
        out_shape=jax.ShapeDtypeStruct((HB, EMB_DIM), jnp.float32),
    )(r, zq, q)


_NW = 32
_BPW = HB // _NW


def _gather_body(table_hbm, idx_hbm, out_hbm, idx_v, rows_v, sem):
    wid = lax.axis_index("s") * 2 + lax.axis_index("c")
    base = wid * _BPW
    pltpu.sync_copy(idx_hbm.at[pl.ds(base, _BPW)], idx_v)
    pltpu.async_copy(table_hbm.at[idx_v], rows_v, sem).wait()
    pltpu.sync_copy(rows_v, out_hbm.at[pl.ds(base, _BPW)])


def _sc_gather(table, idx):
    return pl.kernel(
        _gather_body,
        mesh=plsc.VectorSubcoreMesh(core_axis_name="c", subcore_axis_name="s"),
        out_type=jax.ShapeDtypeStruct((HB, EMB_DIM), jnp.float32),
        scratch_types=[pltpu.VMEM((_BPW,), jnp.int32),
                       pltpu.VMEM((_BPW, EMB_DIM), jnp.float32),
                       pltpu.SemaphoreType.DMA],
    )(table, idx)


def kernel(z, codebooks):
    lin = jnp.zeros((1, 1), jnp.float32)
    preps = [_prep_call(codebooks[l]) for l in range(LEVELS)]
    r = [z[0:HB, :], z[HB:B, :]]
    q = [None, None]
    zq = [None, None]
    idxs = []
    for l in range(LEVELS):
        embbf, esqb = preps[l]
        halves = []
        for h in range(2):
            if l == 0:
                ih, lin = _level0_call(r[h], embbf, esqb, lin)
            else:
                has_q = l >= 2
                ih, rn, qn, lin = _leveln_call(
                    has_q, r[h], zq[h], q[h] if has_q else r[h],
                    embbf, esqb, lin)
                r[h], q[h] = rn, qn
            zq[h] = _sc_gather(codebooks[l], ih.reshape(HB))
            halves.append(ih.reshape(HB))
        idxs.append(jnp.concatenate(halves))
    quant = jnp.concatenate(
        [_final_call(r[h], zq[h], q[h]) for h in range(2)])
    indices = jnp.stack(idxs, axis=0)
    return quant, indices, lin[0, 0]


# speedup vs baseline: 1.2817x; 1.2817x over previous
import functools

import numpy as np
import jax
import jax.numpy as jnp
from jax import lax
from jax.experimental import pallas as pl
from jax.experimental.pallas import tpu as pltpu
from jax.experimental.pallas import tpu_sc as plsc

LEVELS = 4
NUM_EMB = 8192
EMB_DIM = 256
BETA = 0.25
B = 4096

TB = 256
NB = B // TB
KB = 1024
NKC = NUM_EMB // KB
LOSS_SCALE = (1.0 + BETA) / (B * EMB_DIM)
IMAX = np.int32(2**31 - 1)


def _rowsumsq(x):
    s = x[:, 0:128] * x[:, 0:128] + x[:, 128:256] * x[:, 128:256]
    st = jnp.transpose(s)
    acc = st[0:8, :]
    for g in range(1, 16):
        acc = acc + st[8 * g:8 * g + 8, :]
    t1 = acc + pltpu.roll(acc, 4, 0)
    t2 = t1 + pltpu.roll(t1, 6, 0)
    t3 = t2 + pltpu.roll(t2, 7, 0)
    return t3[0:1, :]


def _prep_scratch(cb_ref, embbf_scr, esq_scr):
    cb = cb_ref[:]
    embbf_scr[:] = cb.astype(jnp.bfloat16)
    esq_scr[:] = jnp.broadcast_to(
        jnp.sum(cb * cb, axis=1, keepdims=True), (NUM_EMB, TB))


def _search(r, embbf_scr, esq_scr):
    zsql = _rowsumsq(r)
    zb = jnp.broadcast_to(zsql, (8, TB))
    rb = (r * (-2.0)).astype(jnp.bfloat16)
    colmin = None
    grpsel = None
    for kc in range(NKC):
        eb = embbf_scr[KB * kc:KB * (kc + 1), :]
        mt = lax.dot_general(eb, rb, (((1,), (1,)), ((), ())),
                             preferred_element_type=jnp.float32)
        for r8 in range(KB // 8):
            grp = kc * (KB // 8) + r8
            d = (zb + mt[8 * r8:8 * r8 + 8, :]) \
                + esq_scr[8 * grp:8 * grp + 8, :]
            if colmin is None:
                colmin = d
                grpsel = jnp.zeros((8, TB), jnp.int32)
            else:
                grpsel = jnp.where(d < colmin, jnp.int32(grp), grpsel)
                colmin = jnp.minimum(colmin, d)
    m1 = jnp.minimum(colmin, pltpu.roll(colmin, 4, 0))
    m2 = jnp.minimum(m1, pltpu.roll(m1, 6, 0))
    rowmin = jnp.minimum(m2, pltpu.roll(m2, 7, 0))[0:1, :]
    sub = lax.broadcasted_iota(jnp.int32, (8, TB), 0)
    cand = jnp.where(colmin == rowmin, grpsel * 8 + sub, IMAX)
    c1 = jnp.minimum(cand, pltpu.roll(cand, 4, 0))
    c2 = jnp.minimum(c1, pltpu.roll(c1, 6, 0))
    mini = jnp.minimum(c2, pltpu.roll(c2, 7, 0))[0:1, :]
    return rowmin, mini


def _acc_loss(i, lin_ref, lout_ref, minv):
    part = jnp.sum(minv, axis=1, keepdims=True) * LOSS_SCALE

    @pl.when(i == 0)
    def _():
        lout_ref[:, :] = lin_ref[:, :] + part

    @pl.when(i > 0)
    def _():
        lout_ref[:, :] = lout_ref[:, :] + part


def _level0_body(r_ref, cb_ref, lin_ref, idx_ref, lout_ref,
                 embbf_scr, esq_scr):
    i = pl.program_id(0)

    @pl.when(i == 0)
    def _():
        _prep_scratch(cb_ref, embbf_scr, esq_scr)

    minv, mini = _search(r_ref[:], embbf_scr, esq_scr)
    idx_ref[:] = mini.reshape(1, 1, TB)
    _acc_loss(i, lin_ref, lout_ref, minv)


def _leveln_body(has_q, r_ref, zq_ref, q_ref, cb_ref, lin_ref,
                 idx_ref, rn_ref, qn_ref, lout_ref, embbf_scr, esq_scr):
    i = pl.program_id(0)

    @pl.when(i == 0)
    def _():
        _prep_scratch(cb_ref, embbf_scr, esq_scr)

    r = r_ref[:]
    diff = zq_ref[:] - r
    c = r + diff
    rn = r - c
    qn = (q_ref[:] + c) if has_q else c
    rn_ref[:] = rn
    qn_ref[:] = qn
    minv, mini = _search(rn, embbf_scr, esq_scr)
    idx_ref[:] = mini.reshape(1, 1, TB)
    _acc_loss(i, lin_ref, lout_ref, minv)


_BLK_ROWS = pl.BlockSpec((TB, EMB_DIM), lambda i: (i, 0))
_BLK_IDX = pl.BlockSpec((1, 1, TB), lambda i: (i, 0, 0))
_BLK_CB = pl.BlockSpec((NUM_EMB, EMB_DIM), lambda i: (0, 0))
_BLK_SCALAR = pl.BlockSpec((1, 1), lambda i: (0, 0))
_SCRATCH = [pltpu.VMEM((NUM_EMB, EMB_DIM), jnp.bfloat16),
            pltpu.VMEM((NUM_EMB, TB), jnp.float32)]


def _level0_call(z, cb, lin):
    return pl.pallas_call(
        _level0_body,
        grid=(NB,),
        in_specs=[_BLK_ROWS, _BLK_CB, _BLK_SCALAR],
        out_specs=[_BLK_IDX, _BLK_SCALAR],
        out_shape=[jax.ShapeDtypeStruct((NB, 1, TB), jnp.int32),
                   jax.ShapeDtypeStruct((1, 1), jnp.float32)],
        scratch_shapes=_SCRATCH,
    )(z, cb, lin)


def _leveln_call(has_q, r, zq, q, cb, lin):
    return pl.pallas_call(
        functools.partial(_leveln_body, has_q),
        grid=(NB,),
        in_specs=[_BLK_ROWS, _BLK_ROWS, _BLK_ROWS, _BLK_CB, _BLK_SCALAR],
        out_specs=[_BLK_IDX, _BLK_ROWS, _BLK_ROWS, _BLK_SCALAR],
        out_shape=[jax.ShapeDtypeStruct((NB, 1, TB), jnp.int32),
                   jax.ShapeDtypeStruct((B, EMB_DIM), jnp.float32),
                   jax.ShapeDtypeStruct((B, EMB_DIM), jnp.float32),
                   jax.ShapeDtypeStruct((1, 1), jnp.float32)],
        scratch_shapes=_SCRATCH,
    )(r, zq, q, cb, lin)


def _final_body(r_ref, zq_ref, q_ref, out_ref):
    r = r_ref[:]
    c = r + (zq_ref[:] - r)
    out_ref[:] = q_ref[:] + c


def _final_call(r, zq, q):
    return pl.pallas_call(
        _final_body,
        grid=(NB,),
        in_specs=[_BLK_ROWS, _BLK_ROWS, _BLK_ROWS],
        out_specs=_BLK_ROWS,
        out_shape=jax.ShapeDtypeStruct((B, EMB_DIM), jnp.float32),
    )(r, zq, q)


_NW = 32
_BPW = B // _NW


def _gather_body(table_hbm, idx_hbm, out_hbm, idx_v, rows_v, sem):
    wid = lax.axis_index("s") * 2 + lax.axis_index("c")
    base = wid * _BPW
    pltpu.sync_copy(idx_hbm.at[pl.ds(base, _BPW)], idx_v)
    pltpu.async_copy(table_hbm.at[idx_v], rows_v, sem).wait()
    pltpu.sync_copy(rows_v, out_hbm.at[pl.ds(base, _BPW)])


def _sc_gather(table, idx):
    return pl.kernel(
        _gather_body,
        mesh=plsc.VectorSubcoreMesh(core_axis_name="c", subcore_axis_name="s"),
        out_type=jax.ShapeDtypeStruct((B, EMB_DIM), jnp.float32),
        scratch_types=[pltpu.VMEM((_BPW,), jnp.int32),
                       pltpu.VMEM((_BPW, EMB_DIM), jnp.float32),
                       pltpu.SemaphoreType.DMA],
    )(table, idx)


def kernel(z, codebooks):
    lin = jnp.zeros((1, 1), jnp.float32)
    idx0, lin = _level0_call(z, codebooks[0], lin)
    zq0 = _sc_gather(codebooks[0], idx0.reshape(B))
    idx1, r1, q1, lin = _leveln_call(False, z, zq0, z, codebooks[1], lin)
    zq1 = _sc_gather(codebooks[1], idx1.reshape(B))
    idx2, r2, q2, lin = _leveln_call(True, r1, zq1, q1, codebooks[2], lin)
    zq2 = _sc_gather(codebooks[2], idx2.reshape(B))
    idx3, r3, q3, lin = _leveln_call(True, r2, zq2, q2, codebooks[3], lin)
    zq3 = _sc_gather(codebooks[3], idx3.reshape(B))
    quant = _final_call(r3, zq3, q3)
    indices = jnp.stack([idx0.reshape(B), idx1.reshape(B),
                         idx2.reshape(B), idx3.reshape(B)], axis=0)
    return quant, indices, lin[0, 0]


# speedup vs baseline: 1.2849x; 1.0025x over previous
import functools

import numpy as np
import jax
import jax.numpy as jnp
from jax import lax
from jax.experimental import pallas as pl
from jax.experimental.pallas import tpu as pltpu
from jax.experimental.pallas import tpu_sc as plsc

LEVELS = 4
NUM_EMB = 8192
EMB_DIM = 256
BETA = 0.25
B = 4096

TB = 256
NB = B // TB
KB = 512
NKC = NUM_EMB // KB
LOSS_SCALE = (1.0 + BETA) / (B * EMB_DIM)
IMAX = np.int32(2**31 - 1)


def _rowsumsq(x):
    s = x[:, 0:128] * x[:, 0:128] + x[:, 128:256] * x[:, 128:256]
    st = jnp.transpose(s)
    acc = st[0:8, :]
    for g in range(1, 16):
        acc = acc + st[8 * g:8 * g + 8, :]
    t1 = acc + pltpu.roll(acc, 4, 0)
    t2 = t1 + pltpu.roll(t1, 6, 0)
    t3 = t2 + pltpu.roll(t2, 7, 0)
    return t3[0:1, :]


def _prep_scratch(cb_ref, embbf_scr, esq_scr):
    cb = cb_ref[:]
    embbf_scr[:] = cb.astype(jnp.bfloat16)
    esq_scr[:] = jnp.broadcast_to(
        jnp.sum(cb * cb, axis=1, keepdims=True), (NUM_EMB, TB))


def _search(r, embbf_scr, esq_scr):
    zsql = _rowsumsq(r)
    zb = jnp.broadcast_to(zsql, (8, TB))
    rb = (r * (-2.0)).astype(jnp.bfloat16)
    colmin = None
    grpsel = None
    for kc in range(NKC):
        eb = embbf_scr[KB * kc:KB * (kc + 1), :]
        mt = lax.dot_general(eb, rb, (((1,), (1,)), ((), ())),
                             preferred_element_type=jnp.float32)
        for r8 in range(KB // 8):
            grp = kc * (KB // 8) + r8
            d = (zb + mt[8 * r8:8 * r8 + 8, :]) \
                + esq_scr[8 * grp:8 * grp + 8, :]
            if colmin is None:
                colmin = d
                grpsel = jnp.zeros((8, TB), jnp.int32)
            else:
                grpsel = jnp.where(d < colmin, jnp.int32(grp), grpsel)
                colmin = jnp.minimum(colmin, d)
    m1 = jnp.minimum(colmin, pltpu.roll(colmin, 4, 0))
    m2 = jnp.minimum(m1, pltpu.roll(m1, 6, 0))
    rowmin = jnp.minimum(m2, pltpu.roll(m2, 7, 0))[0:1, :]
    sub = lax.broadcasted_iota(jnp.int32, (8, TB), 0)
    cand = jnp.where(colmin == rowmin, grpsel * 8 + sub, IMAX)
    c1 = jnp.minimum(cand, pltpu.roll(cand, 4, 0))
    c2 = jnp.minimum(c1, pltpu.roll(c1, 6, 0))
    mini = jnp.minimum(c2, pltpu.roll(c2, 7, 0))[0:1, :]
    return rowmin, mini


def _acc_loss(i, lin_ref, lout_ref, minv):
    part = jnp.sum(minv, axis=1, keepdims=True) * LOSS_SCALE

    @pl.when(i == 0)
    def _():
        lout_ref[:, :] = lin_ref[:, :] + part

    @pl.when(i > 0)
    def _():
        lout_ref[:, :] = lout_ref[:, :] + part


def _level0_body(r_ref, cb_ref, lin_ref, idx_ref, lout_ref,
                 embbf_scr, esq_scr):
    i = pl.program_id(0)

    @pl.when(i == 0)
    def _():
        _prep_scratch(cb_ref, embbf_scr, esq_scr)

    minv, mini = _search(r_ref[:], embbf_scr, esq_scr)
    idx_ref[:] = mini.reshape(1, 1, TB)
    _acc_loss(i, lin_ref, lout_ref, minv)


def _leveln_body(has_q, r_ref, zq_ref, q_ref, cb_ref, lin_ref,
                 idx_ref, rn_ref, qn_ref, lout_ref, embbf_scr, esq_scr):
    i = pl.program_id(0)

    @pl.when(i == 0)
    def _():
        _prep_scratch(cb_ref, embbf_scr, esq_scr)

    r = r_ref[:]
    diff = zq_ref[:] - r
    c = r + diff
    rn = r - c
    qn = (q_ref[:] + c) if has_q else c
    rn_ref[:] = rn
    qn_ref[:] = qn
    minv, mini = _search(rn, embbf_scr, esq_scr)
    idx_ref[:] = mini.reshape(1, 1, TB)
    _acc_loss(i, lin_ref, lout_ref, minv)


_BLK_ROWS = pl.BlockSpec((TB, EMB_DIM), lambda i: (i, 0))
_BLK_IDX = pl.BlockSpec((1, 1, TB), lambda i: (i, 0, 0))
_BLK_CB = pl.BlockSpec((NUM_EMB, EMB_DIM), lambda i: (0, 0))
_BLK_SCALAR = pl.BlockSpec((1, 1), lambda i: (0, 0))
_SCRATCH = [pltpu.VMEM((NUM_EMB, EMB_DIM), jnp.bfloat16),
            pltpu.VMEM((NUM_EMB, TB), jnp.float32)]


def _level0_call(z, cb, lin):
    return pl.pallas_call(
        _level0_body,
        grid=(NB,),
        in_specs=[_BLK_ROWS, _BLK_CB, _BLK_SCALAR],
        out_specs=[_BLK_IDX, _BLK_SCALAR],
        out_shape=[jax.ShapeDtypeStruct((NB, 1, TB), jnp.int32),
                   jax.ShapeDtypeStruct((1, 1), jnp.float32)],
        scratch_shapes=_SCRATCH,
    )(z, cb, lin)


def _leveln_call(has_q, r, zq, q, cb, lin):
    return pl.pallas_call(
        functools.partial(_leveln_body, has_q),
        grid=(NB,),
        in_specs=[_BLK_ROWS, _BLK_ROWS, _BLK_ROWS, _BLK_CB, _BLK_SCALAR],
        out_specs=[_BLK_IDX, _BLK_ROWS, _BLK_ROWS, _BLK_SCALAR],
        out_shape=[jax.ShapeDtypeStruct((NB, 1, TB), jnp.int32),
                   jax.ShapeDtypeStruct((B, EMB_DIM), jnp.float32),
                   jax.ShapeDtypeStruct((B, EMB_DIM), jnp.float32),
                   jax.ShapeDtypeStruct((1, 1), jnp.float32)],
        scratch_shapes=_SCRATCH,
    )(r, zq, q, cb, lin)


def _final_body(r_ref, zq_ref, q_ref, out_ref):
    r = r_ref[:]
    c = r + (zq_ref[:] - r)
    out_ref[:] = q_ref[:] + c


def _final_call(r, zq, q):
    return pl.pallas_call(
        _final_body,
        grid=(NB,),
        in_specs=[_BLK_ROWS, _BLK_ROWS, _BLK_ROWS],
        out_specs=_BLK_ROWS,
        out_shape=jax.ShapeDtypeStruct((B, EMB_DIM), jnp.float32),
    )(r, zq, q)


_NW = 32
_BPW = B // _NW


def _gather_body(table_hbm, idx_hbm, out_hbm, idx_v, rows_v, sem):
    wid = lax.axis_index("s") * 2 + lax.axis_index("c")
    base = wid * _BPW
    pltpu.sync_copy(idx_hbm.at[pl.ds(base, _BPW)], idx_v)
    pltpu.async_copy(table_hbm.at[idx_v], rows_v, sem).wait()
    pltpu.sync_copy(rows_v, out_hbm.at[pl.ds(base, _BPW)])


def _sc_gather(table, idx):
    return pl.kernel(
        _gather_body,
        mesh=plsc.VectorSubcoreMesh(core_axis_name="c", subcore_axis_name="s"),
        out_type=jax.ShapeDtypeStruct((B, EMB_DIM), jnp.float32),
        scratch_types=[pltpu.VMEM((_BPW,), jnp.int32),
                       pltpu.VMEM((_BPW, EMB_DIM), jnp.float32),
                       pltpu.SemaphoreType.DMA],
    )(table, idx)


def kernel(z, codebooks):
    lin = jnp.zeros((1, 1), jnp.float32)
    idx0, lin = _level0_call(z, codebooks[0], lin)
    zq0 = _sc_gather(codebooks[0], idx0.reshape(B))
    idx1, r1, q1, lin = _leveln_call(False, z, zq0, z, codebooks[1], lin)
    zq1 = _sc_gather(codebooks[1], idx1.reshape(B))
    idx2, r2, q2, lin = _leveln_call(True, r1, zq1, q1, codebooks[2], lin)
    zq2 = _sc_gather(codebooks[2], idx2.reshape(B))
    idx3, r3, q3, lin = _leveln_call(True, r2, zq2, q2, codebooks[3], lin)
    zq3 = _sc_gather(codebooks[3], idx3.reshape(B))
    quant = _final_call(r3, zq3, q3)
    indices = jnp.stack([idx0.reshape(B), idx1.reshape(B),
                         idx2.reshape(B), idx3.reshape(B)], axis=0)
    return quant, indices, lin[0, 0]
